# baseline (speedup 1.0000x reference)
import functools

import jax
import jax.numpy as jnp
from jax import lax
from jax.experimental import pallas as pl
from jax.experimental.pallas import tpu as pltpu
from jax.experimental.pallas import tpu_sc as plsc

NC = 2
NS = 16
NW = NC * NS
LANES = 16
CHUNK = 320
TROWS = 40


@jax.jit
def _sc_gather(edge_types, table):
    B = edge_types.shape[0]
    V, D = table.shape
    n_chunks = B // CHUNK

    mesh = plsc.VectorSubcoreMesh(
        core_axis_name="c", subcore_axis_name="s", num_cores=NC, num_subcores=NS
    )

    @functools.partial(
        pl.kernel,
        out_type=jax.ShapeDtypeStruct((B, D), jnp.float32),
        mesh=mesh,
        scratch_types=[
            pltpu.VMEM((V * D,), jnp.float32),
            pltpu.VMEM((CHUNK,), jnp.int32),
            pltpu.VMEM((CHUNK,), jnp.int32),
            pltpu.VMEM((CHUNK, D), jnp.float32),
            pltpu.VMEM((CHUNK, D), jnp.float32),
            pltpu.SemaphoreType.DMA,
            pltpu.SemaphoreType.DMA,
            pltpu.SemaphoreType.DMA,
            pltpu.SemaphoreType.DMA,
        ],
        compiler_params=pltpu.CompilerParams(needs_layout_passes=False),
    )
    def k(idx_hbm, table_hbm, out_hbm, tab_flat, i0, i1, r0, r1, si0, si1, sw0, sw1):
        wid = lax.axis_index("s") * NC + lax.axis_index("c")
        idx_v, rows_v = [i0, i1], [r0, r1]
        sem_i, sem_w = [si0, si1], [sw0, sw1]

        count = n_chunks // NW + jnp.where(wid < n_chunks % NW, 1, 0)

        def chunk_off(kk):
            return (wid + kk * NW) * CHUNK

        def start_idx(kk, b):
            pltpu.make_async_copy(
                idx_hbm.at[pl.ds(chunk_off(kk), CHUNK)], idx_v[b], sem_i[b]
            ).start()

        def wait_idx(b):
            pltpu.make_async_copy(
                idx_hbm.at[pl.ds(0, CHUNK)], idx_v[b], sem_i[b]
            ).wait()

        def start_wb(kk, b):
            pltpu.make_async_copy(
                rows_v[b], out_hbm.at[pl.ds(chunk_off(kk), CHUNK)], sem_w[b]
            ).start()

        def wait_wb(b):
            pltpu.make_async_copy(
                rows_v[b], out_hbm.at[pl.ds(0, CHUNK)], sem_w[b]
            ).wait()

        start_idx(0, 0)
        start_idx(1, 1)

        def stage(tmp, tsem):
            for rr0 in range(0, V, TROWS):
                pltpu.make_async_copy(
                    table_hbm.at[pl.ds(rr0, TROWS)], tmp, tsem
                ).start()
                pltpu.make_async_copy(
                    table_hbm.at[pl.ds(rr0, TROWS)], tmp, tsem
                ).wait()

                def crow(r, _):
                    for c in range(0, D, LANES):
                        tab_flat[pl.ds((rr0 + r) * D + c, LANES)] = tmp[
                            r, pl.ds(c, LANES)
                        ]
                    return 0

                lax.fori_loop(0, TROWS, crow, 0)

        pl.run_scoped(
            stage,
            pltpu.VMEM((TROWS, D), jnp.float32),
            pltpu.SemaphoreType.DMA,
        )

        iota = lax.iota(jnp.int32, LANES)
        cols = [jnp.full((LANES,), j, jnp.int32) for j in range(D)]

        def compute(b):
            def grp(g, _):
                row_ids = idx_v[b][pl.ds(g * LANES, LANES)]
                addr = row_ids * D
                lrow = iota + g * LANES
                for j in range(D):
                    v = plsc.load_gather(tab_flat, [addr + j])
                    plsc.store_scatter(rows_v[b], [lrow, cols[j]], v)
                return 0

            lax.fori_loop(0, CHUNK // LANES, grp, 0)

        def do_chunk(kk, b, s):
            wait_idx(b)

            @pl.when(s > 0)
            def _():
                wait_wb(b)

            compute(b)

            @pl.when(kk + 2 < count)
            def _():
                start_idx(kk + 2, b)

            start_wb(kk, b)

        def pair(s, _):
            do_chunk(2 * s, 0, s)
            do_chunk(2 * s + 1, 1, s)
            return 0

        lax.fori_loop(0, count // 2, pair, 0)

        @pl.when(count % 2 == 1)
        def _tail():
            do_chunk(count - 1, 0, count // 2)

        wait_wb(0)
        wait_wb(1)

    return k(edge_types, table)


def kernel(edge_types, table):
    return _sc_gather(edge_types.astype(jnp.int32), table)


# speedup vs baseline: 1.2794x; 1.2794x over previous
import functools

import jax
import jax.numpy as jnp
from jax import lax
from jax.experimental import pallas as pl
from jax.experimental.pallas import tpu as pltpu
from jax.experimental.pallas import tpu_sc as plsc

NC = 2
NS = 16
NW = NC * NS
LANES = 16
CHUNK = 320
TROWS = 40


@jax.jit
def _sc_gather(edge_types, table):
    B = edge_types.shape[0]
    V, D = table.shape
    n_chunks = B // CHUNK

    mesh = plsc.VectorSubcoreMesh(
        core_axis_name="c", subcore_axis_name="s", num_cores=NC, num_subcores=NS
    )

    @functools.partial(
        pl.kernel,
        out_type=jax.ShapeDtypeStruct((B, D), jnp.float32),
        mesh=mesh,
        scratch_types=[
            pltpu.VMEM((V * D,), jnp.float32),
            pltpu.VMEM((CHUNK,), jnp.int32),
            pltpu.VMEM((CHUNK,), jnp.int32),
            pltpu.VMEM((CHUNK, D), jnp.float32),
            pltpu.VMEM((CHUNK, D), jnp.float32),
            pltpu.SemaphoreType.DMA,
            pltpu.SemaphoreType.DMA,
            pltpu.SemaphoreType.DMA,
            pltpu.SemaphoreType.DMA,
        ],
        compiler_params=pltpu.CompilerParams(needs_layout_passes=False),
    )
    def k(idx_hbm, table_hbm, out_hbm, tab_flat, i0, i1, r0, r1, si0, si1, sw0, sw1):
        wid = lax.axis_index("s") * NC + lax.axis_index("c")
        idx_v, rows_v = [i0, i1], [r0, r1]
        sem_i, sem_w = [si0, si1], [sw0, sw1]

        count = n_chunks // NW + jnp.where(wid < n_chunks % NW, 1, 0)

        def chunk_off(kk):
            return (wid + kk * NW) * CHUNK

        def start_idx(kk, b):
            pltpu.make_async_copy(
                idx_hbm.at[pl.ds(chunk_off(kk), CHUNK)], idx_v[b], sem_i[b]
            ).start()

        def wait_idx(b):
            pltpu.make_async_copy(
                idx_hbm.at[pl.ds(0, CHUNK)], idx_v[b], sem_i[b]
            ).wait()

        def start_wb(kk, b):
            pltpu.make_async_copy(
                rows_v[b], out_hbm.at[pl.ds(chunk_off(kk), CHUNK)], sem_w[b]
            ).start()

        def wait_wb(b):
            pltpu.make_async_copy(
                rows_v[b], out_hbm.at[pl.ds(0, CHUNK)], sem_w[b]
            ).wait()

        start_idx(0, 0)
        start_idx(1, 1)

        def stage(tmp, tsem):
            for rr0 in range(0, V, TROWS):
                pltpu.make_async_copy(
                    table_hbm.at[pl.ds(rr0, TROWS)], tmp, tsem
                ).start()
                pltpu.make_async_copy(
                    table_hbm.at[pl.ds(rr0, TROWS)], tmp, tsem
                ).wait()

                def crow(r, _):
                    for c in range(0, D, LANES):
                        tab_flat[pl.ds((rr0 + r) * D + c, LANES)] = tmp[
                            r, pl.ds(c, LANES)
                        ]
                    return 0

                lax.fori_loop(0, TROWS, crow, 0)

        pl.run_scoped(
            stage,
            pltpu.VMEM((TROWS, D), jnp.float32),
            pltpu.SemaphoreType.DMA,
        )

        iota = lax.iota(jnp.int32, LANES)
        cols = [jnp.full((LANES,), j, jnp.int32) for j in range(D)]

        def compute(b):
            def grp(g, _):
                row_ids = idx_v[b][pl.ds(g * LANES, LANES)]
                addr = row_ids * D
                lrow = iota + g * LANES
                vs = [plsc.load_gather(tab_flat, [addr + j]) for j in range(D)]
                for j in range(D):
                    plsc.store_scatter(rows_v[b], [lrow, cols[j]], vs[j])
                return 0

            lax.fori_loop(0, CHUNK // LANES, grp, 0)

        def do_chunk(kk, b, s):
            wait_idx(b)

            @pl.when(s > 0)
            def _():
                wait_wb(b)

            compute(b)

            @pl.when(kk + 2 < count)
            def _():
                start_idx(kk + 2, b)

            start_wb(kk, b)

        def pair(s, _):
            do_chunk(2 * s, 0, s)
            do_chunk(2 * s + 1, 1, s)
            return 0

        lax.fori_loop(0, count // 2, pair, 0)

        @pl.when(count % 2 == 1)
        def _tail():
            do_chunk(count - 1, 0, count // 2)

        wait_wb(0)
        wait_wb(1)

    return k(edge_types, table)


def kernel(edge_types, table):
    return _sc_gather(edge_types.astype(jnp.int32), table)


# speedup vs baseline: 2.5283x; 1.9761x over previous
import functools

import jax
import jax.numpy as jnp
from jax import lax
from jax.experimental import pallas as pl
from jax.experimental.pallas import tpu as pltpu
from jax.experimental.pallas import tpu_sc as plsc

NC = 2
NS = 16
NW = NC * NS
LANES = 16
CHUNK = 320
TROWS = 40


@jax.jit
def _sc_gather(edge_types, table):
    B = edge_types.shape[0]
    V, D = table.shape
    n_chunks = B // CHUNK

    mesh = plsc.VectorSubcoreMesh(
        core_axis_name="c", subcore_axis_name="s", num_cores=NC, num_subcores=NS
    )

    @functools.partial(
        pl.kernel,
        out_type=jax.ShapeDtypeStruct((B, D), jnp.float32),
        mesh=mesh,
        scratch_types=[
            pltpu.VMEM((V * D,), jnp.float32),
            pltpu.VMEM((CHUNK,), jnp.int32),
            pltpu.VMEM((CHUNK,), jnp.int32),
            pltpu.VMEM((CHUNK, D), jnp.float32),
            pltpu.VMEM((CHUNK, D), jnp.float32),
            pltpu.SemaphoreType.DMA,
            pltpu.SemaphoreType.DMA,
            pltpu.SemaphoreType.DMA,
            pltpu.SemaphoreType.DMA,
        ],
        compiler_params=pltpu.CompilerParams(needs_layout_passes=False),
    )
    def k(idx_hbm, table_hbm, out_hbm, tab_flat, i0, i1, r0, r1, si0, si1, sw0, sw1):
        wid = lax.axis_index("s") * NC + lax.axis_index("c")
        idx_v, rows_v = [i0, i1], [r0, r1]
        sem_i, sem_w = [si0, si1], [sw0, sw1]

        count = n_chunks // NW + jnp.where(wid < n_chunks % NW, 1, 0)

        def chunk_off(kk):
            return (wid + kk * NW) * CHUNK

        def start_idx(kk, b):
            pltpu.make_async_copy(
                idx_hbm.at[pl.ds(chunk_off(kk), CHUNK)], idx_v[b], sem_i[b]
            ).start()

        def wait_idx(b):
            pltpu.make_async_copy(
                idx_hbm.at[pl.ds(0, CHUNK)], idx_v[b], sem_i[b]
            ).wait()

        def start_wb(kk, b):
            pltpu.make_async_copy(
                rows_v[b], out_hbm.at[pl.ds(chunk_off(kk), CHUNK)], sem_w[b]
            ).start()

        def wait_wb(b):
            pltpu.make_async_copy(
                rows_v[b], out_hbm.at[pl.ds(0, CHUNK)], sem_w[b]
            ).wait()

        start_idx(0, 0)
        start_idx(1, 1)

        def stage(tmp, tsem):
            for rr0 in range(0, V, TROWS):
                pltpu.make_async_copy(
                    table_hbm.at[pl.ds(rr0, TROWS)], tmp, tsem
                ).start()
                pltpu.make_async_copy(
                    table_hbm.at[pl.ds(rr0, TROWS)], tmp, tsem
                ).wait()

                def crow(r, _):
                    for c in range(0, D, LANES):
                        tab_flat[pl.ds((rr0 + r) * D + c, LANES)] = tmp[
                            r, pl.ds(c, LANES)
                        ]
                    return 0

                lax.fori_loop(0, TROWS, crow, 0)

        pl.run_scoped(
            stage,
            pltpu.VMEM((TROWS, D), jnp.float32),
            pltpu.SemaphoreType.DMA,
        )

        iota = lax.iota(jnp.int32, LANES)
        lane_sel = [jnp.full((LANES, 1), e, jnp.int32) for e in range(LANES)]
        gdn = lax.GatherDimensionNumbers(
            offset_dims=(), collapsed_slice_dims=(0,), start_index_map=(0,)
        )

        def splat(vec, e):
            return lax.gather(
                vec, lane_sel[e], gdn, (1,),
                mode=lax.GatherScatterMode.PROMISE_IN_BOUNDS,
            )

        def compute(b):
            def grp(g, _):
                row_ids = idx_v[b][pl.ds(g * LANES, LANES)]
                bases = row_ids * D
                for e in range(LANES):
                    sp = splat(bases, e)
                    for c in range(0, D, LANES):
                        v = plsc.load_gather(tab_flat, [sp + (iota + c)])
                        rows_v[b][g * LANES + e, pl.ds(c, LANES)] = v
                return 0

            lax.fori_loop(0, CHUNK // LANES, grp, 0)

        def do_chunk(kk, b, s):
            wait_idx(b)

            @pl.when(s > 0)
            def _():
                wait_wb(b)

            compute(b)

            @pl.when(kk + 2 < count)
            def _():
                start_idx(kk + 2, b)

            start_wb(kk, b)

        def pair(s, _):
            do_chunk(2 * s, 0, s)
            do_chunk(2 * s + 1, 1, s)
            return 0

        lax.fori_loop(0, count // 2, pair, 0)

        @pl.when(count % 2 == 1)
        def _tail():
            do_chunk(count - 1, 0, count // 2)

        wait_wb(0)
        wait_wb(1)

    return k(edge_types, table)


def kernel(edge_types, table):
    return _sc_gather(edge_types.astype(jnp.int32), table)


# speedup vs baseline: 9.6304x; 3.8091x over previous
import functools

import jax
import jax.numpy as jnp
from jax import lax
from jax.experimental import pallas as pl
from jax.experimental.pallas import tpu as pltpu
from jax.experimental.pallas import tpu_sc as plsc

NC = 2
NS = 16
NW = NC * NS
LANES = 16
CHUNK = 512


@jax.jit
def _sc_gather(edge_types, table_t):
    B = edge_types.shape[0]
    D, V = table_t.shape
    n_chunks = B // CHUNK

    mesh = plsc.VectorSubcoreMesh(
        core_axis_name="c", subcore_axis_name="s", num_cores=NC, num_subcores=NS
    )

    @functools.partial(
        pl.kernel,
        out_type=jax.ShapeDtypeStruct((D, B), jnp.float32),
        mesh=mesh,
        scratch_types=[
            pltpu.VMEM((D, V), jnp.float32),
            pltpu.VMEM((CHUNK,), jnp.int32),
            pltpu.VMEM((CHUNK,), jnp.int32),
            pltpu.VMEM((D, CHUNK), jnp.float32),
            pltpu.VMEM((D, CHUNK), jnp.float32),
            pltpu.SemaphoreType.DMA,
            pltpu.SemaphoreType.DMA,
            pltpu.SemaphoreType.DMA,
            pltpu.SemaphoreType.DMA,
        ],
        compiler_params=pltpu.CompilerParams(needs_layout_passes=False),
    )
    def k(idx_hbm, tab_hbm, out_hbm, tab_v, i0, i1, r0, r1, si0, si1, sw0, sw1):
        wid = lax.axis_index("s") * NC + lax.axis_index("c")
        idx_v, rows_v = [i0, i1], [r0, r1]
        sem_i, sem_w = [si0, si1], [sw0, sw1]

        count = n_chunks // NW + jnp.where(wid < n_chunks % NW, 1, 0)

        def chunk_off(kk):
            return (wid + kk * NW) * CHUNK

        def start_idx(kk, b):
            pltpu.make_async_copy(
                idx_hbm.at[pl.ds(chunk_off(kk), CHUNK)], idx_v[b], sem_i[b]
            ).start()

        def wait_idx(b):
            pltpu.make_async_copy(
                idx_hbm.at[pl.ds(0, CHUNK)], idx_v[b], sem_i[b]
            ).wait()

        def start_wb(kk, b):
            pltpu.make_async_copy(
                rows_v[b], out_hbm.at[:, pl.ds(chunk_off(kk), CHUNK)], sem_w[b]
            ).start()

        def wait_wb(b):
            pltpu.make_async_copy(
                rows_v[b], out_hbm.at[:, pl.ds(0, CHUNK)], sem_w[b]
            ).wait()

        start_idx(0, 0)
        start_idx(1, 1)
        pltpu.sync_copy(tab_hbm, tab_v)

        cols = [jnp.full((LANES,), j, jnp.int32) for j in range(D)]

        def compute(b):
            def grp(g, _):
                row_ids = idx_v[b][pl.ds(g * LANES, LANES)]
                vs = [plsc.load_gather(tab_v, [cols[j], row_ids]) for j in range(D)]
                for j in range(D):
                    rows_v[b][j, pl.ds(g * LANES, LANES)] = vs[j]
                return 0

            lax.fori_loop(0, CHUNK // LANES, grp, 0)

        def do_chunk(kk, b, s):
            wait_idx(b)

            @pl.when(s > 0)
            def _():
                wait_wb(b)

            compute(b)

            @pl.when(kk + 2 < count)
            def _():
                start_idx(kk + 2, b)

            start_wb(kk, b)

        def pair(s, _):
            do_chunk(2 * s, 0, s)
            do_chunk(2 * s + 1, 1, s)
            return 0

        lax.fori_loop(0, count // 2, pair, 0)

        @pl.when(count % 2 == 1)
        def _tail():
            do_chunk(count - 1, 0, count // 2)

        wait_wb(0)
        wait_wb(1)

    return k(edge_types, table_t)


def kernel(edge_types, table):
    return _sc_gather(edge_types.astype(jnp.int32), table.T).T
